# Initial kernel scaffold; baseline (speedup 1.0000x reference)
import jax
import jax.numpy as jnp
from jax.experimental import pallas as pl


def kernel(x, edge_index, edge_attr, h_node_h, h_node_c, h_edge_h, h_edge_c, W_ih_n, W_hh_n, b_ih_n, b_hh_n, W_ih_e, W_hh_e, b_ih_e, b_hh_e, W_gat, att_src, att_dst, b_gat, W_e1, b_e1, W_e2, b_e2, W_n1, b_n1, W_n2, b_n2):
    raise NotImplementedError("write your pallas kernel here")


# speedup vs baseline: 1.0215x; 1.0215x over previous
import functools

import jax
import jax.numpy as jnp
from jax.experimental import pallas as pl
from jax.experimental.pallas import tpu as pltpu

N = 50000
E = 800000
NF = 16
EF = 16
RNN = 20
RNNE = 8
HID = 64
HEADS = 4
LAT = 32
OUT = 4


def _lstm_block_kernel(x_ref, h_ref, c_ref, wih_ref, whh_ref, b_ref,
                       h_out_ref, c_out_ref):
    x = x_ref[...]
    h = h_ref[...]
    c = c_ref[...]
    gates = (jnp.dot(x, wih_ref[...], preferred_element_type=jnp.float32)
             + jnp.dot(h, whh_ref[...], preferred_element_type=jnp.float32)
             + b_ref[...])
    H = h.shape[1]
    i = jax.nn.sigmoid(gates[:, :H])
    f = jax.nn.sigmoid(gates[:, H:2 * H])
    g = jnp.tanh(gates[:, 2 * H:3 * H])
    o = jax.nn.sigmoid(gates[:, 3 * H:])
    c_new = f * c + i * g
    h_out_ref[...] = o * jnp.tanh(c_new)
    c_out_ref[...] = c_new


def _lstm_pallas(x, h, c, Wih, Whh, bih, bhh, block_rows):
    n, _ = x.shape
    hdim = h.shape[1]
    WihT = Wih.T
    WhhT = Whh.T
    b = (bih + bhh)[None, :]
    grid = (n // block_rows,)
    h_new, c_new = pl.pallas_call(
        _lstm_block_kernel,
        grid=grid,
        in_specs=[
            pl.BlockSpec((block_rows, x.shape[1]), lambda i: (i, 0)),
            pl.BlockSpec((block_rows, hdim), lambda i: (i, 0)),
            pl.BlockSpec((block_rows, hdim), lambda i: (i, 0)),
            pl.BlockSpec((x.shape[1], 4 * hdim), lambda i: (0, 0)),
            pl.BlockSpec((hdim, 4 * hdim), lambda i: (0, 0)),
            pl.BlockSpec((1, 4 * hdim), lambda i: (0, 0)),
        ],
        out_specs=[
            pl.BlockSpec((block_rows, hdim), lambda i: (i, 0)),
            pl.BlockSpec((block_rows, hdim), lambda i: (i, 0)),
        ],
        out_shape=[
            jax.ShapeDtypeStruct((n, hdim), jnp.float32),
            jax.ShapeDtypeStruct((n, hdim), jnp.float32),
        ],
    )(x, h, c, WihT, WhhT, b)
    return h_new, c_new


def kernel(x, edge_index, edge_attr, h_node_h, h_node_c, h_edge_h, h_edge_c,
           W_ih_n, W_hh_n, b_ih_n, b_hh_n,
           W_ih_e, W_hh_e, b_ih_e, b_hh_e,
           W_gat, att_src, att_dst, b_gat,
           W_e1, b_e1, W_e2, b_e2, W_n1, b_n1, W_n2, b_n2):
    src = edge_index[0]
    dst = edge_index[1]
    n = x.shape[0]

    h_e, c_e = _lstm_pallas(edge_attr, h_edge_h, h_edge_c,
                            W_ih_e, W_hh_e, b_ih_e, b_hh_e, block_rows=3200)
    h_n, c_n = _lstm_pallas(x, h_node_h, h_node_c,
                            W_ih_n, W_hh_n, b_ih_n, b_hh_n, block_rows=2000)

    cnt = jax.ops.segment_sum(jnp.ones((dst.shape[0],), jnp.float32), dst,
                              num_segments=n)
    denom_cnt = jnp.maximum(cnt, 1.0)[:, None]
    edge_enc = jax.ops.segment_sum(h_e, dst, num_segments=n) / denom_cnt

    x_enc = jnp.concatenate([h_n, edge_enc], axis=-1)
    xl = (x_enc @ W_gat.T).reshape(n, HEADS, HID)
    a_src = (xl * att_src[None]).sum(-1)
    a_dst = (xl * att_dst[None]).sum(-1)
    alpha = jax.nn.leaky_relu(a_src[src] + a_dst[dst], negative_slope=0.2)
    ex = jnp.exp(alpha)
    sden = jax.ops.segment_sum(ex, dst, num_segments=n)
    attn = ex / (sden[dst] + 1e-16)
    msg = xl[src] * attn[:, :, None]
    x_gat = jax.ops.segment_sum(msg, dst, num_segments=n).mean(axis=1) + b_gat

    e_in = jnp.concatenate([x_gat[src], x_gat[dst], edge_attr], axis=-1)
    e_lat = jax.nn.relu(e_in @ W_e1.T + b_e1) @ W_e2.T + b_e2
    agg = jax.ops.segment_sum(e_lat, dst, num_segments=n) / denom_cnt
    n_in = jnp.concatenate([x_gat, agg], axis=-1)
    out = jax.nn.relu(n_in @ W_n1.T + b_n1) @ W_n2.T + b_n2
    return (out, h_n, c_n, h_e, c_e)


# speedup vs baseline: 18.1667x; 17.7837x over previous
import functools
import os

import jax
import jax.numpy as jnp
from jax import lax
from jax.experimental import pallas as pl
from jax.experimental.pallas import tpu as pltpu
from jax.experimental.pallas import tpu_sc as plsc

try:
    jax.config.update(
        "jax_compilation_cache_dir",
        os.path.join(os.path.dirname(os.path.abspath(__file__)),
                     ".jax_cache"))
    jax.config.update("jax_persistent_cache_min_compile_time_secs", 1.0)
except Exception:
    pass

N = 50000
E = 800000
NF = 16
EF = 16
RNN = 20
RNNE = 8
HID = 64
HEADS = 4
LAT = 32
OUT = 4

_NC = 2
_NS = 16
_NW = _NC * _NS
_EPW = E // _NW
_NPAD = 50176
_RPS = _NPAD // _NS

_SC_MESH = plsc.VectorSubcoreMesh(core_axis_name="c", subcore_axis_name="s")
_SC_PARAMS = pltpu.CompilerParams(use_tc_tiling_on_sc=False)

_NBLK = 2000
_EBLK = 3200


def _tc1e_kernel(ea_ref, h_ref, c_ref, wih_ref, whh_ref, b_ref, cpt_ref,
                 be1_ref, h_out, c_out, hep_out, r_out):
    ea = ea_ref[...]
    h = h_ref[...]
    c = c_ref[...]
    gates = (jnp.dot(ea, wih_ref[...], preferred_element_type=jnp.float32)
             + jnp.dot(h, whh_ref[...], preferred_element_type=jnp.float32)
             + b_ref[...])
    H = RNNE
    i = jax.nn.sigmoid(gates[:, :H])
    f = jax.nn.sigmoid(gates[:, H:2 * H])
    g = jnp.tanh(gates[:, 2 * H:3 * H])
    o = jax.nn.sigmoid(gates[:, 3 * H:])
    c_new = f * c + i * g
    h_new = o * jnp.tanh(c_new)
    h_out[...] = h_new
    c_out[...] = c_new
    b_rows = h_new.shape[0]
    hep_out[...] = jnp.concatenate(
        [h_new, jnp.ones((b_rows, 1), jnp.float32),
         jnp.zeros((b_rows, 16 - H - 1), jnp.float32)], axis=1)
    r_out[...] = jnp.dot(ea, cpt_ref[...],
                         preferred_element_type=jnp.float32) + be1_ref[...]


def _tc1e(edge_attr, h, c, Wih, Whh, bih, bhh, CpT, b_e1):
    grid = (E // _EBLK,)
    bs = lambda w: pl.BlockSpec((_EBLK, w), lambda i: (i, 0))
    full = lambda a: pl.BlockSpec(a.shape, lambda i: (0,) * a.ndim)
    b = (bih + bhh)[None, :]
    be1 = b_e1[None, :]
    return pl.pallas_call(
        _tc1e_kernel,
        grid=grid,
        in_specs=[bs(EF), bs(RNNE), bs(RNNE), full(Wih.T), full(Whh.T),
                  full(b), full(CpT), full(be1)],
        out_specs=[bs(RNNE), bs(RNNE), bs(16), bs(64)],
        out_shape=[jax.ShapeDtypeStruct((E, RNNE), jnp.float32),
                   jax.ShapeDtypeStruct((E, RNNE), jnp.float32),
                   jax.ShapeDtypeStruct((E, 16), jnp.float32),
                   jax.ShapeDtypeStruct((E, 64), jnp.float32)],
    )(edge_attr, h, c, Wih.T, Whh.T, b, CpT, be1)


def _tc1n_kernel(x_ref, h_ref, c_ref, wih_ref, whh_ref, b_ref,
                 h_out, c_out):
    x = x_ref[...]
    h = h_ref[...]
    c = c_ref[...]
    gates = (jnp.dot(x, wih_ref[...], preferred_element_type=jnp.float32)
             + jnp.dot(h, whh_ref[...], preferred_element_type=jnp.float32)
             + b_ref[...])
    H = RNN
    i = jax.nn.sigmoid(gates[:, :H])
    f = jax.nn.sigmoid(gates[:, H:2 * H])
    g = jnp.tanh(gates[:, 2 * H:3 * H])
    o = jax.nn.sigmoid(gates[:, 3 * H:])
    c_new = f * c + i * g
    h_out[...] = o * jnp.tanh(c_new)
    c_out[...] = c_new


def _tc1n(x, h, c, Wih, Whh, bih, bhh):
    grid = (N // _NBLK,)
    bs = lambda w: pl.BlockSpec((_NBLK, w), lambda i: (i, 0))
    full = lambda a: pl.BlockSpec(a.shape, lambda i: (0,) * a.ndim)
    b = (bih + bhh)[None, :]
    return pl.pallas_call(
        _tc1n_kernel,
        grid=grid,
        in_specs=[bs(NF), bs(RNN), bs(RNN), full(Wih.T), full(Whh.T),
                  full(b)],
        out_specs=[bs(RNN), bs(RNN)],
        out_shape=[jax.ShapeDtypeStruct((N, RNN), jnp.float32),
                   jax.ShapeDtypeStruct((N, RNN), jnp.float32)],
    )(x, h, c, Wih.T, Whh.T, b)


def _tc2_kernel(p_ref, hn_ref, wg_ref, as_ref, ad_ref,
                xl_out, ats_out, atd_out, cnt_out):
    psum = p_ref[0] + p_ref[1]
    cnt = psum[:, 8:9]
    dinv = 1.0 / jnp.maximum(cnt, 1.0)
    enc = psum[:, 0:8] * dinv
    xe = jnp.concatenate([hn_ref[...], enc], axis=1)
    xl_out[...] = jnp.dot(xe, wg_ref[...],
                          preferred_element_type=jnp.float32)
    ats_out[...] = jnp.dot(xe, as_ref[...],
                           preferred_element_type=jnp.float32)
    atd_out[...] = jnp.dot(xe, ad_ref[...],
                           preferred_element_type=jnp.float32)
    cnt_out[...] = jnp.broadcast_to(cnt, (cnt.shape[0], 8))


def _tc2(p, h_n, WgT, AsPadT, AdPadT):
    grid = (N // _NBLK,)
    bs = lambda w: pl.BlockSpec((_NBLK, w), lambda i: (i, 0))
    full = lambda a: pl.BlockSpec(a.shape, lambda i: (0,) * a.ndim)
    return pl.pallas_call(
        _tc2_kernel,
        grid=grid,
        in_specs=[pl.BlockSpec((2, _NBLK, 16), lambda i: (0, i, 0)),
                  bs(RNN), full(WgT), full(AsPadT), full(AdPadT)],
        out_specs=[bs(256), bs(16), bs(16), bs(8)],
        out_shape=[jax.ShapeDtypeStruct((N, 256), jnp.float32),
                   jax.ShapeDtypeStruct((N, 16), jnp.float32),
                   jax.ShapeDtypeStruct((N, 16), jnp.float32),
                   jax.ShapeDtypeStruct((N, 8), jnp.float32)],
    )(p, h_n, WgT, AsPadT, AdPadT)


def _tc3_kernel(p_ref, rden_out):
    rden_out[...] = 1.0 / (p_ref[0] + p_ref[1] + 1e-16)


def _tc3(p):
    grid = (N // _NBLK,)
    return pl.pallas_call(
        _tc3_kernel,
        grid=grid,
        in_specs=[pl.BlockSpec((2, _NBLK, 16), lambda i: (0, i, 0))],
        out_specs=pl.BlockSpec((_NBLK, 16), lambda i: (i, 0)),
        out_shape=jax.ShapeDtypeStruct((N, 16), jnp.float32),
    )(p)


def _tc4_kernel(m0_ref, m1_ref, m2_ref, m3_ref, bg_ref, at_ref, bt_ref,
                xg_out, p_out, q_out):
    xg = jnp.concatenate(
        [m0_ref[0] + m0_ref[1], m1_ref[0] + m1_ref[1],
         m2_ref[0] + m2_ref[1], m3_ref[0] + m3_ref[1]],
        axis=1) * (1.0 / HEADS) + bg_ref[...]
    xg_out[...] = xg
    p_out[...] = jnp.dot(xg, at_ref[...], preferred_element_type=jnp.float32)
    q_out[...] = jnp.dot(xg, bt_ref[...], preferred_element_type=jnp.float32)


def _tc4(m0, m1, m2, m3, b_gat, AT, BT):
    grid = (N // _NBLK,)
    bs = lambda w: pl.BlockSpec((_NBLK, w), lambda i: (i, 0))
    full = lambda a: pl.BlockSpec(a.shape, lambda i: (0,) * a.ndim)
    bg = b_gat[None, :]
    pspec = pl.BlockSpec((2, _NBLK, 16), lambda i: (0, i, 0))
    return pl.pallas_call(
        _tc4_kernel,
        grid=grid,
        in_specs=[pspec, pspec, pspec, pspec, full(bg), full(AT), full(BT)],
        out_specs=[bs(64), bs(64), bs(64)],
        out_shape=[jax.ShapeDtypeStruct((N, 64), jnp.float32),
                   jax.ShapeDtypeStruct((N, 64), jnp.float32),
                   jax.ShapeDtypeStruct((N, 64), jnp.float32)],
    )(m0, m1, m2, m3, bg, AT, BT)


def _tc5_kernel(xg_ref, e0_ref, e1_ref, e2_ref, e3_ref, cnt_ref, we2_ref,
                be2_ref, wn1_ref, bn1_ref, wn2_ref, bn2_ref, out_ref):
    s = jnp.concatenate(
        [e0_ref[0] + e0_ref[1], e1_ref[0] + e1_ref[1],
         e2_ref[0] + e2_ref[1], e3_ref[0] + e3_ref[1]], axis=1)
    cnt = cnt_ref[:, 0:1]
    dinv = 1.0 / jnp.maximum(cnt, 1.0)
    u = jnp.minimum(cnt, 1.0)
    agg = jnp.dot(s, we2_ref[...],
                  preferred_element_type=jnp.float32) * dinv + be2_ref[...] * u
    n_in = jnp.concatenate([xg_ref[...], agg], axis=1)
    h1 = jax.nn.relu(jnp.dot(n_in, wn1_ref[...],
                             preferred_element_type=jnp.float32)
                     + bn1_ref[...])
    out_ref[...] = jnp.dot(h1, wn2_ref[...],
                           preferred_element_type=jnp.float32) + bn2_ref[...]


def _tc5(x_gat, e0, e1, e2, e3, cntc, We2T, b_e2, Wn1T, b_n1, Wn2T, b_n2):
    grid = (N // _NBLK,)
    bs = lambda w: pl.BlockSpec((_NBLK, w), lambda i: (i, 0))
    full = lambda a: pl.BlockSpec(a.shape, lambda i: (0,) * a.ndim)
    be2 = b_e2[None, :]
    bn1 = b_n1[None, :]
    bn2 = b_n2[None, :]
    pspec = pl.BlockSpec((2, _NBLK, 16), lambda i: (0, i, 0))
    return pl.pallas_call(
        _tc5_kernel,
        grid=grid,
        in_specs=[bs(64), pspec, pspec, pspec, pspec, bs(8), full(We2T),
                  full(be2), full(Wn1T), full(bn1), full(Wn2T), full(bn2)],
        out_specs=bs(OUT),
        out_shape=jax.ShapeDtypeStruct((N, OUT), jnp.float32),
    )(x_gat, e0, e1, e2, e3, cntc, We2T, be2, Wn1T, bn1, Wn2T, bn2)


_CA = 1000


def _sc_acc_body(dst_hbm, val_hbm, out_hbm, idx_v, rows_v, zbuf, acc):
    c = lax.axis_index("c")
    s = lax.axis_index("s")
    wid = c * _NS + s

    def zb(i, _):
        zbuf[i] = jnp.zeros((16,), jnp.float32)
        return 0
    lax.fori_loop(0, _RPS, zb, 0)
    pltpu.sync_copy(zbuf, acc.at[pl.ds(s * _RPS, _RPS)])
    plsc.subcore_barrier()

    def chunk(i, _):
        base = wid * _EPW + i * _CA
        pltpu.sync_copy(dst_hbm.at[pl.ds(base, _CA)], idx_v)
        pltpu.sync_copy(val_hbm.at[pl.ds(base, _CA)], rows_v)
        pltpu.sync_copy(rows_v, acc.at[idx_v], add=True)
        return 0
    lax.fori_loop(0, _EPW // _CA, chunk, 0)
    plsc.subcore_barrier()
    pltpu.sync_copy(acc.at[pl.ds(s * _RPS, _RPS)],
                    out_hbm.at[c].at[pl.ds(s * _RPS, _RPS)])


_sc_acc = pl.kernel(
    _sc_acc_body,
    out_type=jax.ShapeDtypeStruct((_NC, _NPAD, 16), jnp.float32),
    mesh=_SC_MESH,
    scratch_types=[
        pltpu.VMEM((_CA,), jnp.int32),
        pltpu.VMEM((_CA, 16), jnp.float32),
        pltpu.VMEM((_RPS, 16), jnp.float32),
        pltpu.VMEM_SHARED((_NPAD, 16), jnp.float32),
    ],
    compiler_params=_SC_PARAMS,
)


def _sc_ex_body(src_hbm, dst_hbm, ats_hbm, atd_hbm, ex_hbm,
                idxs_v, idxd_v, gs_v, gd_v, exb_v, sem):
    c = lax.axis_index("c")
    s = lax.axis_index("s")
    wid = c * _NS + s

    def chunk(i, _):
        base = wid * _EPW + i * _CA
        pltpu.sync_copy(src_hbm.at[pl.ds(base, _CA)], idxs_v)
        pltpu.sync_copy(dst_hbm.at[pl.ds(base, _CA)], idxd_v)
        cp1 = pltpu.async_copy(ats_hbm.at[idxs_v], gs_v, sem)
        cp2 = pltpu.async_copy(atd_hbm.at[idxd_v], gd_v, sem)
        cp1.wait()
        cp2.wait()

        def body(k, _):
            v = gs_v[k] + gd_v[k]
            v = jnp.where(v < 0.0, v * 0.2, v)
            exb_v[k] = jnp.exp(v)
            return 0
        lax.fori_loop(0, _CA, body, 0)
        pltpu.sync_copy(exb_v, ex_hbm.at[pl.ds(base, _CA)])
        return 0
    lax.fori_loop(0, _EPW // _CA, chunk, 0)


_sc_ex = pl.kernel(
    _sc_ex_body,
    out_type=jax.ShapeDtypeStruct((E, 16), jnp.float32),
    mesh=_SC_MESH,
    scratch_types=[
        pltpu.VMEM((_CA,), jnp.int32),
        pltpu.VMEM((_CA,), jnp.int32),
        pltpu.VMEM((_CA, 16), jnp.float32),
        pltpu.VMEM((_CA, 16), jnp.float32),
        pltpu.VMEM((_CA, 16), jnp.float32),
        pltpu.SemaphoreType.DMA,
    ],
    compiler_params=_SC_PARAMS,
)


_CM = 200


def _sc_msg_body(src_hbm, dst_hbm, xl_hbm, rden_hbm, ex_hbm,
                 v0_hbm, v1_hbm, v2_hbm, v3_hbm,
                 idxs_v, idxd_v, rows_v, rd_v, exb_v,
                 w0_v, w1_v, w2_v, w3_v, sem):
    c = lax.axis_index("c")
    s = lax.axis_index("s")
    wid = c * _NS + s

    def chunk(i, _):
        base = wid * _EPW + i * _CM
        pltpu.sync_copy(src_hbm.at[pl.ds(base, _CM)], idxs_v)
        pltpu.sync_copy(dst_hbm.at[pl.ds(base, _CM)], idxd_v)
        cp1 = pltpu.async_copy(xl_hbm.at[idxs_v], rows_v, sem)
        cp2 = pltpu.async_copy(rden_hbm.at[idxd_v], rd_v, sem)
        pltpu.sync_copy(ex_hbm.at[pl.ds(base, _CM)], exb_v)
        cp1.wait()
        cp2.wait()

        def body(k, _):
            att = exb_v[k] * rd_v[k]
            outs = []
            for q in range(4):
                a = jnp.zeros((16,), jnp.float32)
                for h in range(HEADS):
                    a = a + att[h] * rows_v[k, pl.ds(64 * h + 16 * q, 16)]
                outs.append(a)
            w0_v[k] = outs[0]
            w1_v[k] = outs[1]
            w2_v[k] = outs[2]
            w3_v[k] = outs[3]
            return 0
        lax.fori_loop(0, _CM, body, 0)
        pltpu.sync_copy(w0_v, v0_hbm.at[pl.ds(base, _CM)])
        pltpu.sync_copy(w1_v, v1_hbm.at[pl.ds(base, _CM)])
        pltpu.sync_copy(w2_v, v2_hbm.at[pl.ds(base, _CM)])
        pltpu.sync_copy(w3_v, v3_hbm.at[pl.ds(base, _CM)])
        return 0
    lax.fori_loop(0, _EPW // _CM, chunk, 0)


_sc_msg = pl.kernel(
    _sc_msg_body,
    out_type=[jax.ShapeDtypeStruct((E, 16), jnp.float32)] * 4,
    mesh=_SC_MESH,
    scratch_types=[
        pltpu.VMEM((_CM,), jnp.int32),
        pltpu.VMEM((_CM,), jnp.int32),
        pltpu.VMEM((_CM, 256), jnp.float32),
        pltpu.VMEM((_CM, 16), jnp.float32),
        pltpu.VMEM((_CM, 16), jnp.float32),
        pltpu.VMEM((_CM, 16), jnp.float32),
        pltpu.VMEM((_CM, 16), jnp.float32),
        pltpu.VMEM((_CM, 16), jnp.float32),
        pltpu.VMEM((_CM, 16), jnp.float32),
        pltpu.SemaphoreType.DMA,
    ],
    compiler_params=_SC_PARAMS,
)


def _sc_epre_body(src_hbm, dst_hbm, p_hbm, q_hbm, r_hbm,
                  v0_hbm, v1_hbm, v2_hbm, v3_hbm,
                  idxs_v, idxd_v, pr_v, qr_v, rr_v,
                  w0_v, w1_v, w2_v, w3_v, sem):
    c = lax.axis_index("c")
    s = lax.axis_index("s")
    wid = c * _NS + s

    def chunk(i, _):
        base = wid * _EPW + i * _CM
        pltpu.sync_copy(src_hbm.at[pl.ds(base, _CM)], idxs_v)
        pltpu.sync_copy(dst_hbm.at[pl.ds(base, _CM)], idxd_v)
        cp1 = pltpu.async_copy(p_hbm.at[idxs_v], pr_v, sem)
        cp2 = pltpu.async_copy(q_hbm.at[idxd_v], qr_v, sem)
        pltpu.sync_copy(r_hbm.at[pl.ds(base, _CM)], rr_v)
        cp1.wait()
        cp2.wait()

        def body(k, _):
            for q, wq in ((0, w0_v), (1, w1_v), (2, w2_v), (3, w3_v)):
                sl = pl.ds(16 * q, 16)
                v = pr_v[k, sl] + qr_v[k, sl] + rr_v[k, sl]
                wq[k] = jnp.maximum(v, 0.0)
            return 0
        lax.fori_loop(0, _CM, body, 0)
        pltpu.sync_copy(w0_v, v0_hbm.at[pl.ds(base, _CM)])
        pltpu.sync_copy(w1_v, v1_hbm.at[pl.ds(base, _CM)])
        pltpu.sync_copy(w2_v, v2_hbm.at[pl.ds(base, _CM)])
        pltpu.sync_copy(w3_v, v3_hbm.at[pl.ds(base, _CM)])
        return 0
    lax.fori_loop(0, _EPW // _CM, chunk, 0)


_sc_epre = pl.kernel(
    _sc_epre_body,
    out_type=[jax.ShapeDtypeStruct((E, 16), jnp.float32)] * 4,
    mesh=_SC_MESH,
    scratch_types=[
        pltpu.VMEM((_CM,), jnp.int32),
        pltpu.VMEM((_CM,), jnp.int32),
        pltpu.VMEM((_CM, 64), jnp.float32),
        pltpu.VMEM((_CM, 64), jnp.float32),
        pltpu.VMEM((_CM, 64), jnp.float32),
        pltpu.VMEM((_CM, 16), jnp.float32),
        pltpu.VMEM((_CM, 16), jnp.float32),
        pltpu.VMEM((_CM, 16), jnp.float32),
        pltpu.VMEM((_CM, 16), jnp.float32),
        pltpu.SemaphoreType.DMA,
    ],
    compiler_params=_SC_PARAMS,
)


def kernel(x, edge_index, edge_attr, h_node_h, h_node_c, h_edge_h, h_edge_c,
           W_ih_n, W_hh_n, b_ih_n, b_hh_n,
           W_ih_e, W_hh_e, b_ih_e, b_hh_e,
           W_gat, att_src, att_dst, b_gat,
           W_e1, b_e1, W_e2, b_e2, W_n1, b_n1, W_n2, b_n2):
    src = edge_index[0]
    dst = edge_index[1]

    As = jnp.einsum("hd,hdk->hk", att_src,
                    W_gat.reshape(HEADS, HID, RNN + RNNE))
    Ad = jnp.einsum("hd,hdk->hk", att_dst,
                    W_gat.reshape(HEADS, HID, RNN + RNNE))
    AsPadT = jnp.concatenate(
        [As, jnp.zeros((16 - HEADS, RNN + RNNE), jnp.float32)], 0).T
    AdPadT = jnp.concatenate(
        [Ad, jnp.zeros((16 - HEADS, RNN + RNNE), jnp.float32)], 0).T
    A = W_e1[:, 0:HID]
    B = W_e1[:, HID:2 * HID]
    CpT = W_e1[:, 2 * HID:].T

    h_e, c_e, he_pad, R = _tc1e(edge_attr, h_edge_h, h_edge_c,
                                W_ih_e, W_hh_e, b_ih_e, b_hh_e, CpT, b_e1)
    h_n, c_n = _tc1n(x, h_node_h, h_node_c, W_ih_n, W_hh_n, b_ih_n, b_hh_n)

    p_enc = _sc_acc(dst, he_pad)

    xl, ats, atd, cntc = _tc2(p_enc, h_n, W_gat.T, AsPadT, AdPadT)

    ex = _sc_ex(src, dst, ats, atd)
    p_sden = _sc_acc(dst, ex)

    rden = _tc3(p_sden)

    v0, v1, v2, v3 = _sc_msg(src, dst, xl, rden, ex)
    m0 = _sc_acc(dst, v0)
    m1 = _sc_acc(dst, v1)
    m2 = _sc_acc(dst, v2)
    m3 = _sc_acc(dst, v3)

    x_gat, P, Q = _tc4(m0, m1, m2, m3, b_gat, A.T, B.T)

    w0, w1, w2, w3 = _sc_epre(src, dst, P, Q, R)
    e0 = _sc_acc(dst, w0)
    e1 = _sc_acc(dst, w1)
    e2 = _sc_acc(dst, w2)
    e3 = _sc_acc(dst, w3)

    out = _tc5(x_gat, e0, e1, e2, e3, cntc, W_e2.T, b_e2, W_n1.T, b_n1,
               W_n2.T, b_n2)
    return (out, h_n, c_n, h_e, c_e)
